# baseline (speedup 1.0000x reference)
import functools

import jax
import jax.numpy as jnp
from jax import lax
from jax.experimental import pallas as pl
from jax.experimental.pallas import tpu as pltpu
from jax.experimental.pallas import tpu_sc as plsc

EMBED_DIM = 64
ROW_PAD = 128
LANES = 16
NUM_WORKERS = 32


def _make_transpose(vocab: int):
    n_full = vocab // ROW_PAD
    tail = vocab - n_full * ROW_PAD
    per_w = n_full // NUM_WORKERS
    extra = n_full - per_w * NUM_WORKERS
    mesh = plsc.VectorSubcoreMesh(core_axis_name="c", subcore_axis_name="s")

    @functools.partial(
        pl.kernel,
        mesh=mesh,
        compiler_params=pltpu.CompilerParams(needs_layout_passes=False),
        out_type=jax.ShapeDtypeStruct((vocab, ROW_PAD), jnp.float32),
        scratch_types=[
            pltpu.VMEM((2, EMBED_DIM, ROW_PAD), jnp.float32),
            pltpu.VMEM((2, ROW_PAD, ROW_PAD), jnp.float32),
            pltpu.VMEM((EMBED_DIM, EMBED_DIM), jnp.float32),
            pltpu.SemaphoreType.DMA((2,)),
            pltpu.SemaphoreType.DMA((2,)),
        ],
    )
    def transpose_kernel(tabt_hbm, out_hbm, cbuf, tbuf, cbuf_t, isem, osem):
        wid = lax.axis_index("s") * 2 + lax.axis_index("c")
        iotas = [lax.iota(jnp.int32, LANES) + g * LANES for g in range(8)]

        def col_of(j):
            return j * NUM_WORKERS + wid

        def in_desc(j, b):
            return pltpu.make_async_copy(
                tabt_hbm.at[:, pl.ds(col_of(j) * ROW_PAD, ROW_PAD)],
                cbuf.at[b],
                isem.at[b],
            )

        def out_desc(j, b):
            return pltpu.make_async_copy(
                tbuf.at[b],
                out_hbm.at[pl.ds(col_of(j) * ROW_PAD, ROW_PAD)],
                osem.at[b],
            )

        def transpose_block(j, b, n_rows):
            def row_body(r, carry):
                col = jnp.full((LANES,), r, jnp.int32)
                for g in range(EMBED_DIM // LANES):
                    v = plsc.load_gather(cbuf.at[b], [iotas[g], col])
                    tbuf[b, r, pl.ds(g * LANES, LANES)] = v
                return carry

            lax.fori_loop(0, n_rows, row_body, 0)

        for b in range(2):
            in_desc(b, b).start()

        def it(i, carry):
            for b in range(2):
                j = 2 * i + b
                in_desc(j, b).wait()

                @pl.when(j >= 2)
                def _():
                    out_desc(j - 2, b).wait()

                transpose_block(j, b, ROW_PAD)
                out_desc(j, b).start()

                @pl.when(j + 2 < per_w)
                def _():
                    in_desc(j + 2, b).start()

            return carry

        lax.fori_loop(0, per_w // 2, it, 0)
        for b in range(2):
            out_desc(per_w - 2 + b, b).wait()

        if extra:
            @pl.when(wid < extra)
            def _():
                base = (n_full - extra + wid) * ROW_PAD
                pltpu.make_async_copy(
                    tabt_hbm.at[:, pl.ds(base, ROW_PAD)], cbuf.at[0], isem.at[0]
                ).start()
                pltpu.make_async_copy(
                    tabt_hbm.at[:, pl.ds(base, ROW_PAD)], cbuf.at[0], isem.at[0]
                ).wait()

                def row_body(r, carry):
                    col = jnp.full((LANES,), r, jnp.int32)
                    for g in range(EMBED_DIM // LANES):
                        v = plsc.load_gather(cbuf.at[0], [iotas[g], col])
                        tbuf[0, r, pl.ds(g * LANES, LANES)] = v
                    return carry

                lax.fori_loop(0, ROW_PAD, row_body, 0)
                pltpu.make_async_copy(
                    tbuf.at[0], out_hbm.at[pl.ds(base, ROW_PAD)], osem.at[0]
                ).start()
                pltpu.make_async_copy(
                    tbuf.at[0], out_hbm.at[pl.ds(base, ROW_PAD)], osem.at[0]
                ).wait()

        if tail:
            @pl.when(wid == extra)
            def _():
                base = n_full * ROW_PAD
                pltpu.make_async_copy(
                    tabt_hbm.at[:, pl.ds(base, tail)], cbuf_t, isem.at[1]
                ).start()
                pltpu.make_async_copy(
                    tabt_hbm.at[:, pl.ds(base, tail)], cbuf_t, isem.at[1]
                ).wait()

                def row_body(r, carry):
                    col = jnp.full((LANES,), r, jnp.int32)
                    for g in range(EMBED_DIM // LANES):
                        v = plsc.load_gather(cbuf_t, [iotas[g], col])
                        tbuf[1, r, pl.ds(g * LANES, LANES)] = v
                    return carry

                lax.fori_loop(0, tail, row_body, 0)
                pltpu.make_async_copy(
                    tbuf.at[1].at[pl.ds(0, tail)],
                    out_hbm.at[pl.ds(base, tail)],
                    osem.at[1],
                ).start()
                pltpu.make_async_copy(
                    tbuf.at[1].at[pl.ds(0, tail)],
                    out_hbm.at[pl.ds(base, tail)],
                    osem.at[1],
                ).wait()

    return transpose_kernel


def _make_gather(batch: int, hist: int, vocab: int):
    bpw = batch // NUM_WORKERS
    nbuf = 4
    mesh = plsc.VectorSubcoreMesh(core_axis_name="c", subcore_axis_name="s")

    @functools.partial(
        pl.kernel,
        mesh=mesh,
        compiler_params=pltpu.CompilerParams(
            use_tc_tiling_on_sc=False, needs_layout_passes=False
        ),
        out_type=jax.ShapeDtypeStruct((hist, EMBED_DIM, batch), jnp.float32),
        scratch_types=[
            pltpu.VMEM((hist, bpw), jnp.int32),
            pltpu.VMEM((nbuf, bpw, ROW_PAD), jnp.float32),
            pltpu.VMEM((2, EMBED_DIM, bpw), jnp.float32),
            pltpu.SemaphoreType.DMA((nbuf,)),
            pltpu.SemaphoreType.DMA((2,)),
        ],
    )
    def gather_kernel(tab_hbm, xt_hbm, out_hbm, idx_v, rows_v, tbuf, gsem, osem):
        wid = lax.axis_index("s") * 2 + lax.axis_index("c")
        b0 = wid * bpw
        iotas = [lax.iota(jnp.int32, LANES) + g * LANES for g in range(8)]
        pltpu.sync_copy(xt_hbm.at[:, pl.ds(b0, bpw)], idx_v)

        def g_desc(h, b):
            return pltpu.make_async_copy(
                tab_hbm.at[idx_v.at[h]], rows_v.at[b], gsem.at[b]
            )

        def o_desc(h, tb):
            return pltpu.make_async_copy(
                tbuf.at[tb], out_hbm.at[h].at[:, pl.ds(b0, bpw)], osem.at[tb]
            )

        def transpose_rows(b, tb):
            def d_body(d, carry):
                row = jnp.full((LANES,), d, jnp.int32)
                for g in range(bpw // LANES):
                    v = plsc.load_gather(rows_v.at[b], [iotas[g], row])
                    tbuf[tb, d, pl.ds(g * LANES, LANES)] = v
                return carry

            lax.fori_loop(0, EMBED_DIM, d_body, 0)

        for b in range(nbuf):
            g_desc(b, b).start()

        def it(i, carry):
            for b in range(nbuf):
                h = nbuf * i + b
                tb = b % 2
                g_desc(h, b).wait()

                @pl.when(h >= 2)
                def _():
                    o_desc(h - 2, tb).wait()

                transpose_rows(b, tb)
                o_desc(h, tb).start()

                @pl.when(h + nbuf < hist)
                def _():
                    g_desc(h + nbuf, b).start()

            return carry

        lax.fori_loop(0, hist // nbuf, it, 0)
        for tb in range(2):
            o_desc(hist - 2 + tb, tb).wait()

    return gather_kernel


def kernel(x, table):
    batch, hist = x.shape
    vocab, _ = table.shape
    table_rm = _make_transpose(vocab)(table.T)
    out = _make_gather(batch, hist, vocab)(table_rm, x.T.astype(jnp.int32))
    return jnp.transpose(out, (2, 0, 1))
